# baseline (speedup 1.0000x reference)
import functools

import jax
import jax.numpy as jnp
from jax.experimental import pallas as pl
from jax.experimental.pallas import tpu as pltpu

N = 50000
E = 800000
IN = 5
H = 64
C = 10
B = 512

_ROWS = 5000


def _mm_stats_body(x_ref, agg_ref, W_ref, b_ref, h_ref, sum_ref, sumsq_ref):
    i = pl.program_id(0)
    u = x_ref[...] + agg_ref[...]
    h = jnp.dot(u, W_ref[...].T, preferred_element_type=jnp.float32) + b_ref[...]
    h_ref[...] = h

    @pl.when(i == 0)
    def _init():
        sum_ref[...] = jnp.zeros_like(sum_ref)
        sumsq_ref[...] = jnp.zeros_like(sumsq_ref)

    sum_ref[...] += jnp.sum(h, axis=0, keepdims=True)
    sumsq_ref[...] += jnp.sum(h * h, axis=0, keepdims=True)


def _norm_body(h_ref, sum_ref, sumsq_ref, g_ref, beta_ref, out_ref):
    mu = sum_ref[...] / N
    var = sumsq_ref[...] / N - mu * mu
    rs = jax.lax.rsqrt(var + 1e-5) * g_ref[...]
    out_ref[...] = jnp.maximum((h_ref[...] - mu) * rs + beta_ref[...], 0.0)


def _dense_block(xprev, agg, W, b, g, beta):
    d = xprev.shape[1]
    nb = N // _ROWS
    h, s, sq = pl.pallas_call(
        _mm_stats_body,
        grid=(nb,),
        in_specs=[
            pl.BlockSpec((_ROWS, d), lambda i: (i, 0)),
            pl.BlockSpec((_ROWS, d), lambda i: (i, 0)),
            pl.BlockSpec((H, d), lambda i: (0, 0)),
            pl.BlockSpec((1, H), lambda i: (0, 0)),
        ],
        out_specs=[
            pl.BlockSpec((_ROWS, H), lambda i: (i, 0)),
            pl.BlockSpec((1, H), lambda i: (0, 0)),
            pl.BlockSpec((1, H), lambda i: (0, 0)),
        ],
        out_shape=[
            jax.ShapeDtypeStruct((N, H), jnp.float32),
            jax.ShapeDtypeStruct((1, H), jnp.float32),
            jax.ShapeDtypeStruct((1, H), jnp.float32),
        ],
    )(xprev, agg, W, b.reshape(1, -1))
    return pl.pallas_call(
        _norm_body,
        grid=(nb,),
        in_specs=[
            pl.BlockSpec((_ROWS, H), lambda i: (i, 0)),
            pl.BlockSpec((1, H), lambda i: (0, 0)),
            pl.BlockSpec((1, H), lambda i: (0, 0)),
            pl.BlockSpec((1, H), lambda i: (0, 0)),
            pl.BlockSpec((1, H), lambda i: (0, 0)),
        ],
        out_specs=pl.BlockSpec((_ROWS, H), lambda i: (i, 0)),
        out_shape=jax.ShapeDtypeStruct((N, H), jnp.float32),
    )(h, s, sq, g.reshape(1, -1), beta.reshape(1, -1))


def _classifier_body(mean_ref, max_ref, W1_ref, b1_ref, W2_ref, b2_ref, out_ref):
    z = jnp.concatenate([mean_ref[...], max_ref[...]], axis=1)
    z = jnp.dot(z, W1_ref[...].T, preferred_element_type=jnp.float32) + b1_ref[...]
    z = jnp.maximum(z, 0.0)
    out_ref[...] = jnp.dot(z, W2_ref[...].T, preferred_element_type=jnp.float32) + b2_ref[...]


def _classifier(mean_pool, max_pool, W1, b1, W2, b2):
    return pl.pallas_call(
        _classifier_body,
        out_shape=jax.ShapeDtypeStruct((B, C), jnp.float32),
    )(mean_pool, max_pool, W1, b1.reshape(1, -1), W2, b2.reshape(1, -1))


def kernel(x, edge_index, edge_attr, batch,
           ee1_W, ee1_b, ee2_W, ee2_b, ee3_W, ee3_b,
           mlp1_W, mlp1_b, bn1_g, bn1_b,
           mlp2_W, mlp2_b, bn2_g, bn2_b,
           mlp3_W, mlp3_b, bn3_g, bn3_b,
           cls1_W, cls1_b, cls2_W, cls2_b):
    src = edge_index[0]
    dst = edge_index[1]
    a = edge_attr[:, 0]

    def edge_pass(h, eW, eb):
        e = a[:, None] * eW[:, 0][None, :] + eb[None, :]
        msg = jax.nn.relu(jnp.take(h, src, axis=0) + e)
        return jax.ops.segment_sum(msg, dst, num_segments=N)

    agg = edge_pass(x, ee1_W, ee1_b)
    h = _dense_block(x, agg, mlp1_W, mlp1_b, bn1_g, bn1_b)
    agg = edge_pass(h, ee2_W, ee2_b)
    h = _dense_block(h, agg, mlp2_W, mlp2_b, bn2_g, bn2_b)
    agg = edge_pass(h, ee3_W, ee3_b)
    h = _dense_block(h, agg, mlp3_W, mlp3_b, bn3_g, bn3_b)

    counts = jax.ops.segment_sum(jnp.ones((N,), jnp.float32), batch, num_segments=B)
    mean_pool = jax.ops.segment_sum(h, batch, num_segments=B) / jnp.maximum(counts, 1.0)[:, None]
    max_pool = jax.ops.segment_max(h, batch, num_segments=B)

    return _classifier(mean_pool, max_pool, cls1_W, cls1_b, cls2_W, cls2_b)


# speedup vs baseline: 3.7665x; 3.7665x over previous
import functools

import jax
import jax.numpy as jnp
from jax import lax
from jax.experimental import pallas as pl
from jax.experimental.pallas import tpu as pltpu
from jax.experimental.pallas import tpu_sc as plsc

N = 50000
E = 800000
IN = 5
H = 64
C = 10
B = 512

NCORE = 2
NSUB = 16
_ROWS = 5000

_MESH = plsc.VectorSubcoreMesh(core_axis_name="c", subcore_axis_name="s",
                               num_cores=NCORE, num_subcores=NSUB)


_EPT = E // NSUB
_K = 128
_NFULL = _EPT // _K
_KR = _EPT - _NFULL * _K


def _edge_pass32_body(h2, esrc, edst, ea, ew, eb, out,
                      acc, rows, rows_r, sidx, didx, sidx_r, didx_r,
                      attr_v, attr_vr, wv, bv, zbuf):
    c = lax.axis_index("c")
    s = lax.axis_index("s")

    def _zb(i, _):
        zbuf[i, pl.ds(0, 16)] = jnp.zeros((16,), jnp.float32)
        zbuf[i, pl.ds(16, 16)] = jnp.zeros((16,), jnp.float32)
        return _
    lax.fori_loop(0, 200, _zb, None)

    def _zero(k, _):
        g = k * NSUB + s

        @pl.when(g < N // 200)
        def _():
            pltpu.sync_copy(zbuf, acc.at[pl.ds(g * 200, 200)])
        return _
    lax.fori_loop(0, (N // 200 + NSUB - 1) // NSUB, _zero, None)
    plsc.subcore_barrier()

    pltpu.sync_copy(ew.at[c], wv)
    pltpu.sync_copy(eb.at[c], bv)
    w0 = wv[pl.ds(0, 16)]
    w1 = wv[pl.ds(16, 16)]
    b0 = bv[pl.ds(0, 16)]
    b1 = bv[pl.ds(16, 16)]
    coff = c * N

    def _chunk(base, k, rows_k, sidx_k, didx_k, attr_k):
        pltpu.sync_copy(esrc.at[pl.ds(base, k)], sidx_k)
        pltpu.sync_copy(edst.at[pl.ds(base, k)], didx_k)
        pltpu.sync_copy(ea.at[pl.ds(base, k)], attr_k)
        for jj in range(k // 16):
            v = sidx_k[pl.ds(jj * 16, 16)]
            sidx_k[pl.ds(jj * 16, 16)] = v + coff
        pltpu.sync_copy(h2.at[sidx_k], rows_k)

        def _edge16(jj, _):
            av_all = attr_k[pl.ds(jj * 16, 16)]
            for e in range(16):
                j = jj * 16 + e
                av = lax.broadcast(av_all[e], (16,))
                r0 = rows_k[j, pl.ds(0, 16)]
                rows_k[j, pl.ds(0, 16)] = jnp.maximum(r0 + (av * w0 + b0), 0.0)
                r1 = rows_k[j, pl.ds(16, 16)]
                rows_k[j, pl.ds(16, 16)] = jnp.maximum(r1 + (av * w1 + b1), 0.0)
            return _
        lax.fori_loop(0, k // 16, _edge16, None)
        pltpu.sync_copy(rows_k, acc.at[didx_k], add=True)

    def _full(kk, _):
        _chunk(s * _EPT + kk * _K, _K, rows, sidx, didx, attr_v)
        return _
    lax.fori_loop(0, _NFULL, _full, None)
    _chunk(s * _EPT + _NFULL * _K, _KR, rows_r, sidx_r, didx_r, attr_vr)

    plsc.subcore_barrier()

    def _wout(k, _):
        g = k * NSUB + s

        @pl.when(g < N // 200)
        def _():
            pltpu.sync_copy(acc.at[pl.ds(g * 200, 200)],
                            out.at[pl.ds(coff + g * 200, 200)])
        return _
    lax.fori_loop(0, (N // 200 + NSUB - 1) // NSUB, _wout, None)


def _edge_pass32(h2, esrc, edst, ea, ew, eb):
    return pl.kernel(
        _edge_pass32_body,
        out_type=jax.ShapeDtypeStruct((2 * N, 32), jnp.float32),
        mesh=_MESH,
        compiler_params=pltpu.CompilerParams(use_tc_tiling_on_sc=False),
        scratch_types=[
            pltpu.VMEM_SHARED((N, 32), jnp.float32),
            pltpu.VMEM((_K, 32), jnp.float32),
            pltpu.VMEM((_KR, 32), jnp.float32),
            pltpu.VMEM((_K,), jnp.int32),
            pltpu.VMEM((_K,), jnp.int32),
            pltpu.VMEM((_KR,), jnp.int32),
            pltpu.VMEM((_KR,), jnp.int32),
            pltpu.VMEM((_K,), jnp.float32),
            pltpu.VMEM((_KR,), jnp.float32),
            pltpu.VMEM((32,), jnp.float32),
            pltpu.VMEM((32,), jnp.float32),
            pltpu.VMEM((200, 32), jnp.float32),
        ],
    )(h2, esrc, edst, ea, ew, eb)


_EPT1 = E // (2 * NSUB)
_NFULL1 = _EPT1 // _K
_KR1 = _EPT1 - _NFULL1 * _K


def _edge_pass16_body(x16, esrc, edst, ea, ew, eb, out,
                      acc, rows, rows_r, sidx, didx, sidx_r, didx_r,
                      attr_v, attr_vr, wv, bv, zbuf):
    c = lax.axis_index("c")
    s = lax.axis_index("s")

    def _zb(i, _):
        zbuf[i, pl.ds(0, 16)] = jnp.zeros((16,), jnp.float32)
        return _
    lax.fori_loop(0, 200, _zb, None)

    def _zero(k, _):
        g = k * NSUB + s

        @pl.when(g < N // 200)
        def _():
            pltpu.sync_copy(zbuf, acc.at[pl.ds(g * 200, 200)])
        return _
    lax.fori_loop(0, (N // 200 + NSUB - 1) // NSUB, _zero, None)
    plsc.subcore_barrier()

    pltpu.sync_copy(ew, wv)
    pltpu.sync_copy(eb, bv)
    w0 = wv[...]
    b0 = bv[...]

    def _chunk(base, k, rows_k, sidx_k, didx_k, attr_k):
        pltpu.sync_copy(esrc.at[pl.ds(base, k)], sidx_k)
        pltpu.sync_copy(edst.at[pl.ds(base, k)], didx_k)
        pltpu.sync_copy(ea.at[pl.ds(base, k)], attr_k)
        pltpu.sync_copy(x16.at[sidx_k], rows_k)

        def _edge16(jj, _):
            av_all = attr_k[pl.ds(jj * 16, 16)]
            for e in range(16):
                j = jj * 16 + e
                av = lax.broadcast(av_all[e], (16,))
                r0 = rows_k[j, pl.ds(0, 16)]
                rows_k[j, pl.ds(0, 16)] = jnp.maximum(r0 + (av * w0 + b0), 0.0)
            return _
        lax.fori_loop(0, k // 16, _edge16, None)
        pltpu.sync_copy(rows_k, acc.at[didx_k], add=True)

    ebase = c * (E // 2) + s * _EPT1

    def _full(kk, _):
        _chunk(ebase + kk * _K, _K, rows, sidx, didx, attr_v)
        return _
    lax.fori_loop(0, _NFULL1, _full, None)
    _chunk(ebase + _NFULL1 * _K, _KR1, rows_r, sidx_r, didx_r, attr_vr)

    plsc.subcore_barrier()

    def _wout(k, _):
        g = k * NSUB + s

        @pl.when(g < N // 200)
        def _():
            pltpu.sync_copy(acc.at[pl.ds(g * 200, 200)],
                            out.at[pl.ds(c * N + g * 200, 200)])
        return _
    lax.fori_loop(0, (N // 200 + NSUB - 1) // NSUB, _wout, None)


def _edge_pass16(x16, esrc, edst, ea, ew, eb):
    return pl.kernel(
        _edge_pass16_body,
        out_type=jax.ShapeDtypeStruct((2 * N, 16), jnp.float32),
        mesh=_MESH,
        compiler_params=pltpu.CompilerParams(use_tc_tiling_on_sc=False),
        scratch_types=[
            pltpu.VMEM_SHARED((N, 16), jnp.float32),
            pltpu.VMEM((_K, 16), jnp.float32),
            pltpu.VMEM((_KR1, 16), jnp.float32),
            pltpu.VMEM((_K,), jnp.int32),
            pltpu.VMEM((_K,), jnp.int32),
            pltpu.VMEM((_KR1,), jnp.int32),
            pltpu.VMEM((_KR1,), jnp.int32),
            pltpu.VMEM((_K,), jnp.float32),
            pltpu.VMEM((_KR1,), jnp.float32),
            pltpu.VMEM((16,), jnp.float32),
            pltpu.VMEM((16,), jnp.float32),
            pltpu.VMEM((200, 16), jnp.float32),
        ],
    )(x16, esrc, edst, ea, ew, eb)


_PCH = 400
_NPCH = N // _PCH


def _pool_body(h2, batch, psum, pmax, sumb, maxb, rowbuf, bid_v):
    c = lax.axis_index("c")
    s = lax.axis_index("s")
    ninf = jnp.full((16,), -jnp.inf, jnp.float32)
    zero = jnp.zeros((16,), jnp.float32)
    one = jnp.ones((16,), jnp.float32)

    def _init(i, _):
        sumb[i, pl.ds(0, 16)] = zero
        sumb[i, pl.ds(16, 16)] = zero
        sumb[i, pl.ds(32, 16)] = zero
        maxb[i, pl.ds(0, 16)] = ninf
        maxb[i, pl.ds(16, 16)] = ninf
        return _
    lax.fori_loop(0, B, _init, None)

    def _chunkloop(k, _):
        g = k * NSUB + s

        @pl.when(g < _NPCH)
        def _():
            pltpu.sync_copy(h2.at[pl.ds(c * N + g * _PCH, _PCH)], rowbuf)
            pltpu.sync_copy(batch.at[pl.ds(g * _PCH, _PCH)], bid_v)

            def _row16(jj, _):
                bids = bid_v[pl.ds(jj * 16, 16)]
                for e in range(16):
                    j = jj * 16 + e
                    bid = bids[e]
                    r0 = rowbuf[j, pl.ds(0, 16)]
                    r1 = rowbuf[j, pl.ds(16, 16)]
                    plsc.addupdate(sumb.at[bid, pl.ds(0, 16)], r0)
                    plsc.addupdate(sumb.at[bid, pl.ds(16, 16)], r1)
                    plsc.addupdate(sumb.at[bid, pl.ds(32, 16)], one)
                    m0 = maxb[bid, pl.ds(0, 16)]
                    maxb[bid, pl.ds(0, 16)] = jnp.maximum(m0, r0)
                    m1 = maxb[bid, pl.ds(16, 16)]
                    maxb[bid, pl.ds(16, 16)] = jnp.maximum(m1, r1)
                return _
            lax.fori_loop(0, _PCH // 16, _row16, None)
        return _
    lax.fori_loop(0, (_NPCH + NSUB - 1) // NSUB, _chunkloop, None)

    w = c * NSUB + s
    pltpu.sync_copy(sumb, psum.at[w])
    pltpu.sync_copy(maxb, pmax.at[w])


def _pool(h2, batch):
    return pl.kernel(
        _pool_body,
        out_type=[
            jax.ShapeDtypeStruct((2 * NSUB, B, 48), jnp.float32),
            jax.ShapeDtypeStruct((2 * NSUB, B, 32), jnp.float32),
        ],
        mesh=_MESH,
        compiler_params=pltpu.CompilerParams(use_tc_tiling_on_sc=False),
        scratch_types=[
            pltpu.VMEM((B, 48), jnp.float32),
            pltpu.VMEM((B, 32), jnp.float32),
            pltpu.VMEM((_PCH, 32), jnp.float32),
            pltpu.VMEM((_PCH,), jnp.int32),
        ],
    )(h2, batch)


def _mm_stats16_body(x_ref, alo_ref, ahi_ref, W_ref, b_ref,
                     h_ref, sum_ref, sumsq_ref):
    i = pl.program_id(1)
    u = x_ref[...] + alo_ref[...] + ahi_ref[...]
    h = jnp.dot(u, W_ref[...].T, preferred_element_type=jnp.float32) + b_ref[0]
    h_ref[...] = h

    @pl.when(i == 0)
    def _init():
        sum_ref[...] = jnp.zeros_like(sum_ref)
        sumsq_ref[...] = jnp.zeros_like(sumsq_ref)

    sum_ref[...] += jnp.sum(h, axis=0, keepdims=True)[None]
    sumsq_ref[...] += jnp.sum(h * h, axis=0, keepdims=True)[None]


def _mm_stats64_body(xlo_ref, xhi_ref, alo_ref, ahi_ref, W_ref, b_ref,
                     h_ref, sum_ref, sumsq_ref):
    i = pl.program_id(1)
    u = jnp.concatenate([xlo_ref[...] + alo_ref[...],
                         xhi_ref[...] + ahi_ref[...]], axis=1)
    h = jnp.dot(u, W_ref[...].T, preferred_element_type=jnp.float32) + b_ref[0]
    h_ref[...] = h

    @pl.when(i == 0)
    def _init():
        sum_ref[...] = jnp.zeros_like(sum_ref)
        sumsq_ref[...] = jnp.zeros_like(sumsq_ref)

    sum_ref[...] += jnp.sum(h, axis=0, keepdims=True)[None]
    sumsq_ref[...] += jnp.sum(h * h, axis=0, keepdims=True)[None]


def _norm_body(h_ref, sum_ref, sumsq_ref, g_ref, beta_ref, out_ref):
    mu = sum_ref[0] / N
    var = sumsq_ref[0] / N - mu * mu
    rs = lax.rsqrt(var + 1e-5) * g_ref[0]
    out_ref[...] = jnp.maximum((h_ref[...] - mu) * rs + beta_ref[0], 0.0)


_NB = N // _ROWS


def _mm_stats_outs():
    return dict(
        out_specs=[
            pl.BlockSpec((_ROWS, 32), lambda j, i: (j * _NB + i, 0)),
            pl.BlockSpec((1, 1, 32), lambda j, i: (j, 0, 0)),
            pl.BlockSpec((1, 1, 32), lambda j, i: (j, 0, 0)),
        ],
        out_shape=[
            jax.ShapeDtypeStruct((2 * N, 32), jnp.float32),
            jax.ShapeDtypeStruct((2, 1, 32), jnp.float32),
            jax.ShapeDtypeStruct((2, 1, 32), jnp.float32),
        ],
    )


def _norm_split(h, s, sq, g, beta):
    return pl.pallas_call(
        _norm_body,
        grid=(2, _NB),
        in_specs=[
            pl.BlockSpec((_ROWS, 32), lambda j, i: (j * _NB + i, 0)),
            pl.BlockSpec((1, 1, 32), lambda j, i: (j, 0, 0)),
            pl.BlockSpec((1, 1, 32), lambda j, i: (j, 0, 0)),
            pl.BlockSpec((1, 1, 32), lambda j, i: (j, 0, 0)),
            pl.BlockSpec((1, 1, 32), lambda j, i: (j, 0, 0)),
        ],
        out_specs=pl.BlockSpec((_ROWS, 32), lambda j, i: (j * _NB + i, 0)),
        out_shape=jax.ShapeDtypeStruct((2 * N, 32), jnp.float32),
    )(h, s, sq, g.reshape(2, 1, 32), beta.reshape(2, 1, 32))


def _dense16(x16, agg1, W16, b, g, beta):
    h, s, sq = pl.pallas_call(
        _mm_stats16_body,
        grid=(2, _NB),
        in_specs=[
            pl.BlockSpec((_ROWS, 16), lambda j, i: (i, 0)),
            pl.BlockSpec((_ROWS, 16), lambda j, i: (i, 0)),
            pl.BlockSpec((_ROWS, 16), lambda j, i: (_NB + i, 0)),
            pl.BlockSpec((32, 16), lambda j, i: (j, 0)),
            pl.BlockSpec((1, 1, 32), lambda j, i: (j, 0, 0)),
        ],
        **_mm_stats_outs(),
    )(x16, agg1, agg1, W16, b.reshape(2, 1, 32))
    return _norm_split(h, s, sq, g, beta)


def _dense64(h2, agg2, W, b, g, beta):
    h, s, sq = pl.pallas_call(
        _mm_stats64_body,
        grid=(2, _NB),
        in_specs=[
            pl.BlockSpec((_ROWS, 32), lambda j, i: (i, 0)),
            pl.BlockSpec((_ROWS, 32), lambda j, i: (_NB + i, 0)),
            pl.BlockSpec((_ROWS, 32), lambda j, i: (i, 0)),
            pl.BlockSpec((_ROWS, 32), lambda j, i: (_NB + i, 0)),
            pl.BlockSpec((32, H), lambda j, i: (j, 0)),
            pl.BlockSpec((1, 1, 32), lambda j, i: (j, 0, 0)),
        ],
        **_mm_stats_outs(),
    )(h2, h2, agg2, agg2, W, b.reshape(2, 1, 32))
    return _norm_split(h, s, sq, g, beta)


def _final_body(psum_ref, pmax_ref, W1_ref, b1_ref, W2_ref, b2_ref, out_ref):
    P = psum_ref[...]
    S0 = jnp.sum(P[:NSUB], axis=0)
    S1 = jnp.sum(P[NSUB:], axis=0)
    cnt = S0[:, 32:33]
    mean = jnp.concatenate([S0[:, :32], S1[:, :32]], axis=1) / jnp.maximum(cnt, 1.0)
    M = pmax_ref[...]
    mx = jnp.concatenate([jnp.max(M[:NSUB], axis=0), jnp.max(M[NSUB:], axis=0)],
                         axis=1)
    z = jnp.concatenate([mean, mx], axis=1)
    z = jnp.dot(z, W1_ref[...].T, preferred_element_type=jnp.float32) + b1_ref[...]
    z = jnp.maximum(z, 0.0)
    out_ref[...] = jnp.dot(z, W2_ref[...].T,
                           preferred_element_type=jnp.float32) + b2_ref[...]


def _final(psum, pmax, W1, b1, W2, b2):
    return pl.pallas_call(
        _final_body,
        out_shape=jax.ShapeDtypeStruct((B, C), jnp.float32),
    )(psum, pmax, W1, b1.reshape(1, -1), W2, b2.reshape(1, -1))


def kernel(x, edge_index, edge_attr, batch,
           ee1_W, ee1_b, ee2_W, ee2_b, ee3_W, ee3_b,
           mlp1_W, mlp1_b, bn1_g, bn1_b,
           mlp2_W, mlp2_b, bn2_g, bn2_b,
           mlp3_W, mlp3_b, bn3_g, bn3_b,
           cls1_W, cls1_b, cls2_W, cls2_b):
    ea = edge_attr.reshape(E)
    x16 = jnp.pad(x, ((0, 0), (0, 16 - IN)))
    W116 = jnp.pad(mlp1_W, ((0, 0), (0, 16 - IN)))
    w1e = jnp.pad(ee1_W[:, 0], (0, 16 - IN))
    b1e = jnp.pad(ee1_b, (0, 16 - IN))
    ew2 = ee2_W[:, 0].reshape(2, 32)
    eb2 = ee2_b.reshape(2, 32)
    ew3 = ee3_W[:, 0].reshape(2, 32)
    eb3 = ee3_b.reshape(2, 32)

    esrc = edge_index[0]
    edst = edge_index[1]
    agg1 = _edge_pass16(x16, esrc, edst, ea, w1e, b1e)
    h2 = _dense16(x16, agg1, W116, mlp1_b, bn1_g, bn1_b)

    agg2 = _edge_pass32(h2, esrc, edst, ea, ew2, eb2)
    h2 = _dense64(h2, agg2, mlp2_W, mlp2_b, bn2_g, bn2_b)

    agg3 = _edge_pass32(h2, esrc, edst, ea, ew3, eb3)
    h2 = _dense64(h2, agg3, mlp3_W, mlp3_b, bn3_g, bn3_b)

    psum, pmax = _pool(h2, batch)
    return _final(psum, pmax, cls1_W, cls1_b, cls2_W, cls2_b)
